# baseline (speedup 1.0000x reference)
import jax
import jax.numpy as jnp
from jax.experimental import pallas as pl

_BLK_S = 256
_EPS = 1e-5


def _ln_body(x_ref, pos_ref, g_ref, b_ref, o_ref):
    h = x_ref[0] + pos_ref[...]
    mean = jnp.mean(h, axis=-1, keepdims=True)
    d = h - mean
    var = jnp.mean(d * d, axis=-1, keepdims=True)
    o_ref[0] = d * jax.lax.rsqrt(var + _EPS) * g_ref[...] + b_ref[...]


def kernel(x, pos_table, gamma, beta):
    B, S, D = x.shape
    gamma2 = gamma.reshape(1, D)
    beta2 = beta.reshape(1, D)
    grid = (S // _BLK_S, B)
    return pl.pallas_call(
        _ln_body,
        grid=grid,
        in_specs=[
            pl.BlockSpec((1, _BLK_S, D), lambda s, b: (b, s, 0)),
            pl.BlockSpec((_BLK_S, D), lambda s, b: (s, 0)),
            pl.BlockSpec((1, D), lambda s, b: (0, 0)),
            pl.BlockSpec((1, D), lambda s, b: (0, 0)),
        ],
        out_specs=pl.BlockSpec((1, _BLK_S, D), lambda s, b: (b, s, 0)),
        out_shape=jax.ShapeDtypeStruct((B, S, D), x.dtype),
    )(x, pos_table, gamma2, beta2)


# speedup vs baseline: 1.5020x; 1.5020x over previous
import jax
import jax.numpy as jnp
from jax.experimental import pallas as pl

_BLK_S = 1024
_EPS = 1e-5


def _ln_body(x_ref, pos_ref, g_ref, b_ref, o_ref):
    h = x_ref[0] + pos_ref[...]
    mean = jnp.mean(h, axis=-1, keepdims=True)
    d = h - mean
    var = jnp.mean(d * d, axis=-1, keepdims=True)
    o_ref[0] = d * jax.lax.rsqrt(var + _EPS) * g_ref[...] + b_ref[...]


def kernel(x, pos_table, gamma, beta):
    B, S, D = x.shape
    gamma2 = gamma.reshape(1, D)
    beta2 = beta.reshape(1, D)
    grid = (S // _BLK_S, B)
    return pl.pallas_call(
        _ln_body,
        grid=grid,
        in_specs=[
            pl.BlockSpec((1, _BLK_S, D), lambda s, b: (b, s, 0)),
            pl.BlockSpec((_BLK_S, D), lambda s, b: (s, 0)),
            pl.BlockSpec((1, D), lambda s, b: (0, 0)),
            pl.BlockSpec((1, D), lambda s, b: (0, 0)),
        ],
        out_specs=pl.BlockSpec((1, _BLK_S, D), lambda s, b: (b, s, 0)),
        out_shape=jax.ShapeDtypeStruct((B, S, D), x.dtype),
    )(x, pos_table, gamma2, beta2)


# speedup vs baseline: 1.5904x; 1.0589x over previous
import jax
import jax.numpy as jnp
from jax.experimental import pallas as pl

_BLK_S = 2048
_EPS = 1e-5


def _ln_body(x_ref, pos_ref, g_ref, b_ref, o_ref):
    h = x_ref[0] + pos_ref[...]
    mean = jnp.mean(h, axis=-1, keepdims=True)
    d = h - mean
    var = jnp.mean(d * d, axis=-1, keepdims=True)
    o_ref[0] = d * jax.lax.rsqrt(var + _EPS) * g_ref[...] + b_ref[...]


def kernel(x, pos_table, gamma, beta):
    B, S, D = x.shape
    gamma2 = gamma.reshape(1, D)
    beta2 = beta.reshape(1, D)
    grid = (S // _BLK_S, B)
    return pl.pallas_call(
        _ln_body,
        grid=grid,
        in_specs=[
            pl.BlockSpec((1, _BLK_S, D), lambda s, b: (b, s, 0)),
            pl.BlockSpec((_BLK_S, D), lambda s, b: (s, 0)),
            pl.BlockSpec((1, D), lambda s, b: (0, 0)),
            pl.BlockSpec((1, D), lambda s, b: (0, 0)),
        ],
        out_specs=pl.BlockSpec((1, _BLK_S, D), lambda s, b: (b, s, 0)),
        out_shape=jax.ShapeDtypeStruct((B, S, D), x.dtype),
    )(x, pos_table, gamma2, beta2)


# speedup vs baseline: 1.6973x; 1.0672x over previous
import jax
import jax.numpy as jnp
from jax.experimental import pallas as pl

_BLK_S = 512
_EPS = 1e-5


def _ln_body(x_ref, pos_ref, g_ref, b_ref, o_ref):
    h = x_ref[...] + pos_ref[...]
    mean = jnp.mean(h, axis=-1, keepdims=True)
    d = h - mean
    var = jnp.mean(d * d, axis=-1, keepdims=True)
    o_ref[...] = d * jax.lax.rsqrt(var + _EPS) * g_ref[...] + b_ref[...]


def kernel(x, pos_table, gamma, beta):
    B, S, D = x.shape
    gamma2 = gamma.reshape(1, 1, D)
    beta2 = beta.reshape(1, 1, D)
    grid = (S // _BLK_S,)
    return pl.pallas_call(
        _ln_body,
        grid=grid,
        in_specs=[
            pl.BlockSpec((B, _BLK_S, D), lambda s: (0, s, 0)),
            pl.BlockSpec((1, _BLK_S, D), lambda s: (0, s, 0)),
            pl.BlockSpec((1, 1, D), lambda s: (0, 0, 0)),
            pl.BlockSpec((1, 1, D), lambda s: (0, 0, 0)),
        ],
        out_specs=pl.BlockSpec((B, _BLK_S, D), lambda s: (0, s, 0)),
        out_shape=jax.ShapeDtypeStruct((B, S, D), x.dtype),
    )(x, pos_table.reshape(1, S, D), gamma2, beta2)
